# baseline (speedup 1.0000x reference)
import functools

import jax
import jax.numpy as jnp
from jax import lax
from jax.experimental import pallas as pl
from jax.experimental.pallas import tpu as pltpu
from jax.experimental.pallas import tpu_sc as plsc

NUM_SEG = 1024
BLK = 2000
WIN = 64
CH = 64
NSPLIT = 2
NC = 2
NS = 16
NW = NC * NS
SPW = NUM_SEG // NW
LANES = 16


def _dense_body(starts_ref, fb_ref, x_ref, batch_ref, w_ref, b_ref,
                part_ref, fbacc_ref):
    bidx = pl.program_id(0)

    @pl.when(bidx == 0)
    def _():
        fbacc_ref[...] = jnp.zeros_like(fbacc_ref)

    xb = x_ref[...]
    xh = xb.astype(jnp.bfloat16)
    s = lax.dot_general(xh, w_ref[...], (((1,), (0,)), ((), ())),
                        preferred_element_type=jnp.float32)
    wgt = jax.nn.sigmoid(s + b_ref[0, 0])
    xwh = (xb * wgt).astype(jnp.bfloat16)

    brow = batch_ref[0]
    start = starts_ref[bidx]
    fb = fb_ref[bidx]

    @pl.when(fb == 0)
    def _():
        col = lax.broadcasted_iota(jnp.int32, (WIN, BLK), 0) + start
        oh = (col == brow).astype(jnp.bfloat16)
        part_ref[0] = lax.dot_general(oh, xwh, (((1,), (0,)), ((), ())),
                                      preferred_element_type=jnp.float32)

    @pl.when(fb != 0)
    def _():
        part_ref[0] = jnp.zeros_like(part_ref[0])
        col = lax.broadcasted_iota(jnp.int32, (NUM_SEG, BLK), 0)
        oh = (col == brow).astype(jnp.bfloat16)
        fbacc_ref[...] += lax.dot_general(oh, xwh, (((1,), (0,)), ((), ())),
                                          preferred_element_type=jnp.float32)


def _add_body(a_ref, b_ref, o_ref):
    o_ref[...] = a_ref[...] + b_ref[...]


def kernel(x, batch, W, b):
    n, f = x.shape
    nb = n // BLK
    nb_h = nb // NSPLIT

    firsts = batch[::BLK]
    lasts = batch[BLK - 1::BLK]
    starts = jnp.minimum(firsts, NUM_SEG - WIN).astype(jnp.int32)
    fb = (lasts >= starts + WIN).astype(jnp.int32)
    batch3 = batch.reshape(nb, 1, BLK)
    Wh = W.astype(jnp.bfloat16)
    b2 = b.reshape(1, 1)

    mesh = plsc.VectorSubcoreMesh(core_axis_name="c", subcore_axis_name="s")

    @functools.partial(
        pl.kernel,
        out_type=jax.ShapeDtypeStruct((NUM_SEG, f), jnp.float32),
        mesh=mesh,
        scratch_types=[
            pltpu.VMEM((LANES,), jnp.int32),
            pltpu.VMEM((CH,), jnp.int32),
            pltpu.VMEM((CH, LANES), jnp.int32),
            pltpu.VMEM((CH, f), jnp.float32),
            pltpu.VMEM((SPW, f), jnp.float32),
        ],
        compiler_params=pltpu.CompilerParams(use_tc_tiling_on_sc=False,
                                             needs_layout_passes=False),
    )
    def _sc_merge(parts_hbm, fbacc_hbm, idx_hbm, segs_hbm, bounds_hbm,
                  out_hbm, bv, rowidx_v, segsp_v, buf, acc):
        c = lax.axis_index("c")
        sid = lax.axis_index("s")
        w = sid * NC + c
        base_seg = w * SPW
        out_off = pl.multiple_of(w * SPW, 8)
        iot = lax.iota(jnp.int32, LANES)

        pltpu.sync_copy(fbacc_hbm.at[pl.ds(out_off, SPW)], acc)

        pltpu.sync_copy(bounds_hbm.at[pl.ds(pl.multiple_of(w * LANES, 8),
                                            LANES)], bv)
        bvec = bv[...]
        n_chunks = jnp.sum(jnp.where(iot == 0, bvec, 0))
        begin_al = jnp.sum(jnp.where(iot == 1, bvec, 0))

        def chunk_body(k, carry):
            off = pl.multiple_of(begin_al + k * CH, 8)
            pltpu.sync_copy(idx_hbm.at[pl.ds(off, CH)], rowidx_v)
            pltpu.sync_copy(segs_hbm.at[pl.ds(off, CH)], segsp_v)
            pltpu.sync_copy(parts_hbm.at[rowidx_v], buf)
            for r in range(CH):
                loc = segsp_v[r] - base_seg
                msk = (loc >= 0) & (loc < SPW)
                for cb in range(f // LANES):
                    cvec = iot + cb * LANES
                    vals = buf[r, pl.ds(cb * LANES, LANES)]
                    plsc.addupdate_scatter(acc, [loc, cvec], vals, mask=msk)
            return carry

        lax.fori_loop(0, n_chunks, chunk_body, 0)

        pltpu.sync_copy(acc, out_hbm.at[pl.ds(out_off, SPW)])

    def stage(h):
        sl = slice(h * nb_h, (h + 1) * nb_h)
        starts_h = starts[sl]
        fb_h = fb[sl]
        nrow_h = nb_h * WIN

        grid_spec = pltpu.PrefetchScalarGridSpec(
            num_scalar_prefetch=2,
            grid=(nb_h,),
            in_specs=[
                pl.BlockSpec((BLK, f), lambda i, *_, h=h: (i + h * nb_h, 0)),
                pl.BlockSpec((1, 1, BLK),
                             lambda i, *_, h=h: (i + h * nb_h, 0, 0)),
                pl.BlockSpec((f, 1), lambda i, *_: (0, 0)),
                pl.BlockSpec((1, 1), lambda i, *_: (0, 0)),
            ],
            out_specs=[
                pl.BlockSpec((1, WIN, f), lambda i, *_: (i, 0, 0)),
                pl.BlockSpec((NUM_SEG, f), lambda i, *_: (0, 0)),
            ],
        )
        parts, fbacc = pl.pallas_call(
            _dense_body,
            grid_spec=grid_spec,
            out_shape=[
                jax.ShapeDtypeStruct((nb_h, WIN, f), jnp.float32),
                jax.ShapeDtypeStruct((NUM_SEG, f), jnp.float32),
            ],
            compiler_params=pltpu.CompilerParams(
                dimension_semantics=("arbitrary",)),
        )(starts_h, fb_h, x, batch3, Wh, b2)

        parts2 = parts.reshape(nrow_h, f)

        seg_of_row = (starts_h[:, None]
                      + jnp.arange(WIN, dtype=jnp.int32)[None, :]).reshape(-1)
        order = jnp.argsort(seg_of_row).astype(jnp.int32)
        segs_sorted = seg_of_row[order]
        idx_pad = jnp.concatenate([order, jnp.zeros((2 * CH,), jnp.int32)])
        segs_pad = jnp.concatenate(
            [segs_sorted, jnp.full((2 * CH,), 2 * NUM_SEG, jnp.int32)])
        segs_splat = jnp.broadcast_to(segs_pad[:, None],
                                      (segs_pad.shape[0], LANES))
        ws = jnp.arange(NW, dtype=jnp.int32)
        lo = jnp.searchsorted(segs_sorted, ws * SPW).astype(jnp.int32)
        hi = jnp.searchsorted(segs_sorted, ws * SPW + SPW).astype(jnp.int32)
        begin = lo // 8 * 8
        nch = (hi - begin + CH - 1) // CH
        bounds = jnp.zeros((NW, LANES), jnp.int32)
        bounds = bounds.at[:, 0].set(nch).at[:, 1].set(begin).reshape(-1)

        return _sc_merge(parts2, fbacc, idx_pad, segs_splat, bounds)

    outs = [stage(h) for h in range(NSPLIT)]
    return pl.pallas_call(
        _add_body,
        out_shape=jax.ShapeDtypeStruct((NUM_SEG, f), jnp.float32),
    )(outs[0], outs[1])


# speedup vs baseline: 1.1414x; 1.1414x over previous
import functools

import jax
import jax.numpy as jnp
from jax import lax
from jax.experimental import pallas as pl
from jax.experimental.pallas import tpu as pltpu
from jax.experimental.pallas import tpu_sc as plsc

NUM_SEG = 1024
BLK = 1000
WIN = 32
CH = 64
NSPLIT = 1
NC = 2
NS = 16
NW = NC * NS
SPW = NUM_SEG // NW
LANES = 16


def _dense_body(starts_ref, fb_ref, x_ref, batch_ref, w_ref, b_ref,
                part_ref, fbacc_ref):
    bidx = pl.program_id(0)

    @pl.when(bidx == 0)
    def _():
        fbacc_ref[...] = jnp.zeros_like(fbacc_ref)

    xb = x_ref[...]
    xh = xb.astype(jnp.bfloat16)
    s = lax.dot_general(xh, w_ref[...], (((1,), (0,)), ((), ())),
                        preferred_element_type=jnp.float32)
    wgt = jax.nn.sigmoid(s + b_ref[0, 0])
    xwh = (xb * wgt).astype(jnp.bfloat16)

    brow = batch_ref[0]
    start = starts_ref[bidx]
    fb = fb_ref[bidx]

    @pl.when(fb == 0)
    def _():
        col = lax.broadcasted_iota(jnp.int32, (WIN, BLK), 0) + start
        oh = (col == brow).astype(jnp.bfloat16)
        part_ref[0] = lax.dot_general(oh, xwh, (((1,), (0,)), ((), ())),
                                      preferred_element_type=jnp.float32)

    @pl.when(fb != 0)
    def _():
        part_ref[0] = jnp.zeros_like(part_ref[0])
        col = lax.broadcasted_iota(jnp.int32, (NUM_SEG, BLK), 0)
        oh = (col == brow).astype(jnp.bfloat16)
        fbacc_ref[...] += lax.dot_general(oh, xwh, (((1,), (0,)), ((), ())),
                                          preferred_element_type=jnp.float32)


def _add_body(a_ref, b_ref, o_ref):
    o_ref[...] = a_ref[...] + b_ref[...]


def kernel(x, batch, W, b):
    n, f = x.shape
    nb = n // BLK
    nb_h = nb // NSPLIT

    firsts = batch[::BLK]
    lasts = batch[BLK - 1::BLK]
    starts = jnp.minimum(firsts, NUM_SEG - WIN).astype(jnp.int32)
    fb = (lasts >= starts + WIN).astype(jnp.int32)
    batch3 = batch.reshape(nb, 1, BLK)
    Wh = W.astype(jnp.bfloat16)
    b2 = b.reshape(1, 1)

    mesh = plsc.VectorSubcoreMesh(core_axis_name="c", subcore_axis_name="s")

    @functools.partial(
        pl.kernel,
        out_type=jax.ShapeDtypeStruct((NUM_SEG, f), jnp.float32),
        mesh=mesh,
        scratch_types=[
            pltpu.VMEM((LANES,), jnp.int32),
            pltpu.VMEM((CH,), jnp.int32),
            pltpu.VMEM((CH, LANES), jnp.int32),
            pltpu.VMEM((CH, f), jnp.float32),
            pltpu.VMEM((SPW, f), jnp.float32),
        ],
        compiler_params=pltpu.CompilerParams(use_tc_tiling_on_sc=False,
                                             needs_layout_passes=False),
    )
    def _sc_merge(parts_hbm, fbacc_hbm, idx_hbm, segs_hbm, bounds_hbm,
                  out_hbm, bv, rowidx_v, segsp_v, buf, acc):
        c = lax.axis_index("c")
        sid = lax.axis_index("s")
        w = sid * NC + c
        base_seg = w * SPW
        out_off = pl.multiple_of(w * SPW, 8)
        iot = lax.iota(jnp.int32, LANES)

        pltpu.sync_copy(fbacc_hbm.at[pl.ds(out_off, SPW)], acc)

        pltpu.sync_copy(bounds_hbm.at[pl.ds(pl.multiple_of(w * LANES, 8),
                                            LANES)], bv)
        bvec = bv[...]
        n_chunks = jnp.sum(jnp.where(iot == 0, bvec, 0))
        begin_al = jnp.sum(jnp.where(iot == 1, bvec, 0))

        def chunk_body(k, carry):
            off = pl.multiple_of(begin_al + k * CH, 8)
            pltpu.sync_copy(idx_hbm.at[pl.ds(off, CH)], rowidx_v)
            pltpu.sync_copy(segs_hbm.at[pl.ds(off, CH)], segsp_v)
            pltpu.sync_copy(parts_hbm.at[rowidx_v], buf)
            for r in range(CH):
                loc = segsp_v[r] - base_seg
                msk = (loc >= 0) & (loc < SPW)
                for cb in range(f // LANES):
                    cvec = iot + cb * LANES
                    vals = buf[r, pl.ds(cb * LANES, LANES)]
                    plsc.addupdate_scatter(acc, [loc, cvec], vals, mask=msk)
            return carry

        lax.fori_loop(0, n_chunks, chunk_body, 0)

        pltpu.sync_copy(acc, out_hbm.at[pl.ds(out_off, SPW)])

    def stage(h):
        sl = slice(h * nb_h, (h + 1) * nb_h)
        starts_h = starts[sl]
        fb_h = fb[sl]
        nrow_h = nb_h * WIN

        grid_spec = pltpu.PrefetchScalarGridSpec(
            num_scalar_prefetch=2,
            grid=(nb_h,),
            in_specs=[
                pl.BlockSpec((BLK, f), lambda i, *_, h=h: (i + h * nb_h, 0)),
                pl.BlockSpec((1, 1, BLK),
                             lambda i, *_, h=h: (i + h * nb_h, 0, 0)),
                pl.BlockSpec((f, 1), lambda i, *_: (0, 0)),
                pl.BlockSpec((1, 1), lambda i, *_: (0, 0)),
            ],
            out_specs=[
                pl.BlockSpec((1, WIN, f), lambda i, *_: (i, 0, 0)),
                pl.BlockSpec((NUM_SEG, f), lambda i, *_: (0, 0)),
            ],
        )
        parts, fbacc = pl.pallas_call(
            _dense_body,
            grid_spec=grid_spec,
            out_shape=[
                jax.ShapeDtypeStruct((nb_h, WIN, f), jnp.float32),
                jax.ShapeDtypeStruct((NUM_SEG, f), jnp.float32),
            ],
            compiler_params=pltpu.CompilerParams(
                dimension_semantics=("arbitrary",)),
        )(starts_h, fb_h, x, batch3, Wh, b2)

        parts2 = parts.reshape(nrow_h, f)

        seg_of_row = (starts_h[:, None]
                      + jnp.arange(WIN, dtype=jnp.int32)[None, :]).reshape(-1)
        order = jnp.argsort(seg_of_row).astype(jnp.int32)
        segs_sorted = seg_of_row[order]
        idx_pad = jnp.concatenate([order, jnp.zeros((2 * CH,), jnp.int32)])
        segs_pad = jnp.concatenate(
            [segs_sorted, jnp.full((2 * CH,), 2 * NUM_SEG, jnp.int32)])
        segs_splat = jnp.broadcast_to(segs_pad[:, None],
                                      (segs_pad.shape[0], LANES))
        ws = jnp.arange(NW, dtype=jnp.int32)
        lo = jnp.searchsorted(segs_sorted, ws * SPW).astype(jnp.int32)
        hi = jnp.searchsorted(segs_sorted, ws * SPW + SPW).astype(jnp.int32)
        begin = lo // 8 * 8
        nch = (hi - begin + CH - 1) // CH
        bounds = jnp.zeros((NW, LANES), jnp.int32)
        bounds = bounds.at[:, 0].set(nch).at[:, 1].set(begin).reshape(-1)

        return _sc_merge(parts2, fbacc, idx_pad, segs_splat, bounds)

    outs = [stage(h) for h in range(NSPLIT)]
    if len(outs) == 1:
        return outs[0]
    return pl.pallas_call(
        _add_body,
        out_shape=jax.ShapeDtypeStruct((NUM_SEG, f), jnp.float32),
    )(outs[0], outs[1])


# speedup vs baseline: 1.4684x; 1.2864x over previous
import functools

import jax
import jax.numpy as jnp
from jax import lax
from jax.experimental import pallas as pl
from jax.experimental.pallas import tpu as pltpu
from jax.experimental.pallas import tpu_sc as plsc

NUM_SEG = 1024
BLK = 4000
WIN = 64
CH = 64
NSPLIT = 1
NC = 2
NS = 16
NW = NC * NS
SPW = NUM_SEG // NW
LANES = 16


def _dense_body(starts_ref, fb_ref, x_ref, batch_ref, w_ref, b_ref,
                part_ref, fbacc_ref):
    bidx = pl.program_id(0)

    @pl.when(bidx == 0)
    def _():
        fbacc_ref[...] = jnp.zeros_like(fbacc_ref)

    xb = x_ref[...]
    xh = xb.astype(jnp.bfloat16)
    s = lax.dot_general(xh, w_ref[...], (((1,), (0,)), ((), ())),
                        preferred_element_type=jnp.float32)
    wgt = jax.nn.sigmoid(s + b_ref[0, 0])
    xwh = (xb * wgt).astype(jnp.bfloat16)

    brow = batch_ref[0]
    start = starts_ref[bidx]
    fb = fb_ref[bidx]

    @pl.when(fb == 0)
    def _():
        col = lax.broadcasted_iota(jnp.int32, (WIN, BLK), 0) + start
        oh = (col == brow).astype(jnp.bfloat16)
        part_ref[0] = lax.dot_general(oh, xwh, (((1,), (0,)), ((), ())),
                                      preferred_element_type=jnp.float32)

    @pl.when(fb != 0)
    def _():
        part_ref[0] = jnp.zeros_like(part_ref[0])
        col = lax.broadcasted_iota(jnp.int32, (NUM_SEG, BLK), 0)
        oh = (col == brow).astype(jnp.bfloat16)
        fbacc_ref[...] += lax.dot_general(oh, xwh, (((1,), (0,)), ((), ())),
                                          preferred_element_type=jnp.float32)


def _add_body(a_ref, b_ref, o_ref):
    o_ref[...] = a_ref[...] + b_ref[...]


def kernel(x, batch, W, b):
    n, f = x.shape
    nb = n // BLK
    nb_h = nb // NSPLIT

    firsts = batch[::BLK]
    lasts = batch[BLK - 1::BLK]
    starts = jnp.minimum(firsts, NUM_SEG - WIN).astype(jnp.int32)
    fb = (lasts >= starts + WIN).astype(jnp.int32)
    batch3 = batch.reshape(nb, 1, BLK)
    Wh = W.astype(jnp.bfloat16)
    b2 = b.reshape(1, 1)

    mesh = plsc.VectorSubcoreMesh(core_axis_name="c", subcore_axis_name="s")

    @functools.partial(
        pl.kernel,
        out_type=jax.ShapeDtypeStruct((NUM_SEG, f), jnp.float32),
        mesh=mesh,
        scratch_types=[
            pltpu.VMEM((LANES,), jnp.int32),
            pltpu.VMEM((CH,), jnp.int32),
            pltpu.VMEM((CH, LANES), jnp.int32),
            pltpu.VMEM((CH, f), jnp.float32),
            pltpu.VMEM((SPW, f), jnp.float32),
        ],
        compiler_params=pltpu.CompilerParams(use_tc_tiling_on_sc=False,
                                             needs_layout_passes=False),
    )
    def _sc_merge(parts_hbm, fbacc_hbm, idx_hbm, segs_hbm, bounds_hbm,
                  out_hbm, bv, rowidx_v, segsp_v, buf, acc):
        c = lax.axis_index("c")
        sid = lax.axis_index("s")
        w = sid * NC + c
        base_seg = w * SPW
        out_off = pl.multiple_of(w * SPW, 8)
        iot = lax.iota(jnp.int32, LANES)

        pltpu.sync_copy(fbacc_hbm.at[pl.ds(out_off, SPW)], acc)

        pltpu.sync_copy(bounds_hbm.at[pl.ds(pl.multiple_of(w * LANES, 8),
                                            LANES)], bv)
        bvec = bv[...]
        n_chunks = jnp.sum(jnp.where(iot == 0, bvec, 0))
        begin_al = jnp.sum(jnp.where(iot == 1, bvec, 0))

        def chunk_body(k, carry):
            off = pl.multiple_of(begin_al + k * CH, 8)
            pltpu.sync_copy(idx_hbm.at[pl.ds(off, CH)], rowidx_v)
            pltpu.sync_copy(segs_hbm.at[pl.ds(off, CH)], segsp_v)
            pltpu.sync_copy(parts_hbm.at[rowidx_v], buf)
            for r in range(CH):
                loc = segsp_v[r] - base_seg
                msk = (loc >= 0) & (loc < SPW)
                for cb in range(f // LANES):
                    cvec = iot + cb * LANES
                    vals = buf[r, pl.ds(cb * LANES, LANES)]
                    plsc.addupdate_scatter(acc, [loc, cvec], vals, mask=msk)
            return carry

        lax.fori_loop(0, n_chunks, chunk_body, 0)

        pltpu.sync_copy(acc, out_hbm.at[pl.ds(out_off, SPW)])

    def stage(h):
        sl = slice(h * nb_h, (h + 1) * nb_h)
        starts_h = starts[sl]
        fb_h = fb[sl]
        nrow_h = nb_h * WIN

        grid_spec = pltpu.PrefetchScalarGridSpec(
            num_scalar_prefetch=2,
            grid=(nb_h,),
            in_specs=[
                pl.BlockSpec((BLK, f), lambda i, *_, h=h: (i + h * nb_h, 0)),
                pl.BlockSpec((1, 1, BLK),
                             lambda i, *_, h=h: (i + h * nb_h, 0, 0)),
                pl.BlockSpec((f, 1), lambda i, *_: (0, 0)),
                pl.BlockSpec((1, 1), lambda i, *_: (0, 0)),
            ],
            out_specs=[
                pl.BlockSpec((1, WIN, f), lambda i, *_: (i, 0, 0)),
                pl.BlockSpec((NUM_SEG, f), lambda i, *_: (0, 0)),
            ],
        )
        parts, fbacc = pl.pallas_call(
            _dense_body,
            grid_spec=grid_spec,
            out_shape=[
                jax.ShapeDtypeStruct((nb_h, WIN, f), jnp.float32),
                jax.ShapeDtypeStruct((NUM_SEG, f), jnp.float32),
            ],
            compiler_params=pltpu.CompilerParams(
                dimension_semantics=("arbitrary",)),
        )(starts_h, fb_h, x, batch3, Wh, b2)

        parts2 = parts.reshape(nrow_h, f)

        seg_of_row = (starts_h[:, None]
                      + jnp.arange(WIN, dtype=jnp.int32)[None, :]).reshape(-1)
        order = jnp.argsort(seg_of_row).astype(jnp.int32)
        segs_sorted = seg_of_row[order]
        idx_pad = jnp.concatenate([order, jnp.zeros((2 * CH,), jnp.int32)])
        segs_pad = jnp.concatenate(
            [segs_sorted, jnp.full((2 * CH,), 2 * NUM_SEG, jnp.int32)])
        segs_splat = jnp.broadcast_to(segs_pad[:, None],
                                      (segs_pad.shape[0], LANES))
        ws = jnp.arange(NW, dtype=jnp.int32)
        lo = jnp.searchsorted(segs_sorted, ws * SPW).astype(jnp.int32)
        hi = jnp.searchsorted(segs_sorted, ws * SPW + SPW).astype(jnp.int32)
        begin = lo // 8 * 8
        nch = (hi - begin + CH - 1) // CH
        bounds = jnp.zeros((NW, LANES), jnp.int32)
        bounds = bounds.at[:, 0].set(nch).at[:, 1].set(begin).reshape(-1)

        return _sc_merge(parts2, fbacc, idx_pad, segs_splat, bounds)

    outs = [stage(h) for h in range(NSPLIT)]
    if len(outs) == 1:
        return outs[0]
    return pl.pallas_call(
        _add_body,
        out_shape=jax.ShapeDtypeStruct((NUM_SEG, f), jnp.float32),
    )(outs[0], outs[1])
